# baseline (speedup 1.0000x reference)
import functools

import jax
import jax.numpy as jnp
from jax import lax
from jax.experimental import pallas as pl
from jax.experimental.pallas import tpu as pltpu
from jax.experimental.pallas import tpu_sc as plsc

N = 10000
E = 320000
NUM_REL = 5
DIM = 128
HALF = DIM // 2
RN = NUM_REL * N
RN_PAD = 51200

NUM_TILES = 32
EPT = E // NUM_TILES
SUP = 2000
SUB = 80
N_PER_SUB = N // 16

_mesh = plsc.VectorSubcoreMesh(core_axis_name="c", subcore_axis_name="s")


def _leaky(x):
    return jnp.where(x > 0, x, 0.01 * x)


@functools.partial(
    pl.kernel,
    out_type=[jax.ShapeDtypeStruct((RN_PAD,), jnp.float32),
              jax.ShapeDtypeStruct((RN_PAD,), jnp.float32)],
    mesh=_mesh,
    compiler_params=pltpu.CompilerParams(needs_layout_passes=False),
    scratch_types=[
        pltpu.VMEM_SHARED((RN_PAD,), jnp.float32),
        pltpu.VMEM((SUP,), jnp.int32),
        pltpu.VMEM((SUP,), jnp.int32),
        pltpu.VMEM((SUB,), jnp.int32),
        pltpu.VMEM((SUB,), jnp.float32),
        pltpu.VMEM((3200,), jnp.float32),
    ],
)
def _sc_counts(dst_hbm, typ_hbm, out0_hbm, out1_hbm, acc_sh, dst_v, typ_v,
               key_v, one_v, buf_v):
    c = lax.axis_index("c")
    sid = lax.axis_index("s")
    wid = sid * 2 + c

    def zbuf(i, _):
        buf_v[pl.ds(i * 16, 16)] = jnp.zeros((16,), jnp.float32)
        return _
    lax.fori_loop(0, 200, zbuf, None)
    pltpu.sync_copy(buf_v, acc_sh.at[pl.ds(sid * 3200, 3200)])

    def ones(i, _):
        one_v[pl.ds(i * 16, 16)] = jnp.ones((16,), jnp.float32)
        return _
    lax.fori_loop(0, SUB // 16, ones, None)
    plsc.subcore_barrier()

    ebase = wid * EPT

    def super_body(sc, _):
        base = ebase + sc * SUP
        pltpu.sync_copy(dst_hbm.at[pl.ds(base, SUP)], dst_v)
        pltpu.sync_copy(typ_hbm.at[pl.ds(base, SUP)], typ_v)

        def sub_body(m, _):
            def key_body(j, _):
                off = m * SUB + j * 16
                d16 = dst_v[pl.ds(off, 16)]
                t16 = typ_v[pl.ds(off, 16)]
                key_v[pl.ds(j * 16, 16)] = t16 * N + d16
                return _
            lax.fori_loop(0, SUB // 16, key_body, None)
            pltpu.sync_copy(one_v, acc_sh.at[key_v], add=True)
            return _
        lax.fori_loop(0, SUP // SUB, sub_body, None)
        return _
    lax.fori_loop(0, EPT // SUP, super_body, None)
    plsc.subcore_barrier()

    pltpu.sync_copy(acc_sh.at[pl.ds(sid * 3200, 3200)], buf_v)

    @pl.when(c == 0)
    def _():
        pltpu.sync_copy(buf_v, out0_hbm.at[pl.ds(sid * 3200, 3200)])

    @pl.when(c == 1)
    def _():
        pltpu.sync_copy(buf_v, out1_hbm.at[pl.ds(sid * 3200, 3200)])


NBUF = 4


def _agg_scratch():
    per_buf = []
    for _ in range(NBUF):
        per_buf += [
            pltpu.VMEM((SUB,), jnp.int32),
            pltpu.VMEM((SUB,), jnp.int32),
            pltpu.VMEM((SUB,), jnp.int32),
            pltpu.VMEM((SUB,), jnp.float32),
            pltpu.VMEM((SUB, DIM), jnp.float32),
            pltpu.SemaphoreType.DMA,
            pltpu.SemaphoreType.DMA,
        ]
    return [
        pltpu.VMEM_SHARED((N, DIM), jnp.float32),
        pltpu.VMEM((SUP,), jnp.int32),
        pltpu.VMEM((SUP,), jnp.int32),
        pltpu.VMEM((SUP,), jnp.int32),
    ] + per_buf


@functools.partial(
    pl.kernel,
    out_type=[jax.ShapeDtypeStruct((N, DIM), jnp.float32),
              jax.ShapeDtypeStruct((N, DIM), jnp.float32)],
    mesh=_mesh,
    compiler_params=pltpu.CompilerParams(needs_layout_passes=False),
    scratch_types=_agg_scratch(),
)
def _sc_agg(zt_hbm, inv_hbm, src_hbm, dst_hbm, typ_hbm, out0_hbm, out1_hbm,
            acc_sh, src_v, dst_v, typ_v, *bufflat):
    c = lax.axis_index("c")
    sid = lax.axis_index("s")
    wid = sid * 2 + c
    bufs = tuple(bufflat[i * 7:(i + 1) * 7] for i in range(NBUF))
    rows_z = bufs[0][4]

    def zrow16(i, _):
        rows_z[i // 8, pl.ds((i % 8) * 16, 16)] = jnp.zeros((16,), jnp.float32)
        return _
    lax.fori_loop(0, SUB * 8, zrow16, None)

    def zacc(i, _):
        k = sid + i * 16

        @pl.when(k < N // SUB)
        def _():
            pltpu.sync_copy(rows_z, acc_sh.at[pl.ds(k * SUB, SUB)])
        return _
    lax.fori_loop(0, 8, zacc, None)
    plsc.subcore_barrier()

    ebase = wid * EPT
    n_chunks = SUP // SUB

    def wait_scatter(b):
        _, _, d_v, _, rows_v, _, ssem = bufs[b]
        pltpu.make_async_copy(rows_v, acc_sh.at[d_v], ssem).wait()

    def prep_start(m, b):
        g_v, k_v, d_v, w_v, rows_v, gsem, _ = bufs[b]

        def idx_body(j, _):
            off = m * SUB + j * 16
            s16 = src_v[pl.ds(off, 16)]
            d16 = dst_v[pl.ds(off, 16)]
            t16 = typ_v[pl.ds(off, 16)]
            g_v[pl.ds(j * 16, 16)] = t16 * N + s16
            d_v[pl.ds(j * 16, 16)] = d16
            k_v[pl.ds(j * 16, 16)] = t16 * N + d16
            return _
        lax.fori_loop(0, SUB // 16, idx_body, None)
        pltpu.async_copy(inv_hbm.at[k_v], w_v, gsem)
        pltpu.async_copy(zt_hbm.at[g_v], rows_v, gsem)

    def process(b):
        g_v, k_v, d_v, w_v, rows_v, gsem, ssem = bufs[b]
        pltpu.make_async_copy(inv_hbm.at[k_v], w_v, gsem).wait()
        pltpu.make_async_copy(zt_hbm.at[g_v], rows_v, gsem).wait()

        def scale_body(ii, _):
            for rr in range(4):
                i = ii * 4 + rr
                wb = plsc.load_gather(w_v, [jnp.broadcast_to(i, (16,))])
                for jj in range(DIM // 16):
                    sl = pl.ds(jj * 16, 16)
                    rows_v[i, sl] = rows_v[i, sl] * wb
            return _
        lax.fori_loop(0, SUB // 4, scale_body, None)
        pltpu.async_copy(rows_v, acc_sh.at[d_v], ssem, add=True)

    for sup in range(EPT // SUP):
        base = ebase + sup * SUP
        pltpu.sync_copy(src_hbm.at[pl.ds(base, SUP)], src_v)
        pltpu.sync_copy(dst_hbm.at[pl.ds(base, SUP)], dst_v)
        pltpu.sync_copy(typ_hbm.at[pl.ds(base, SUP)], typ_v)

        for b in range(NBUF - 1):
            if sup > 0:
                wait_scatter(b)
            prep_start(b, b)

        def pipe_body(j, _, _first_sup=(sup == 0)):
            for rr in range(4):
                m = 4 * j + rr
                process(rr)
                nb = (rr + 3) % 4
                if _first_sup and rr == 0:
                    @pl.when(j > 0)
                    def _():
                        wait_scatter(nb)
                else:
                    wait_scatter(nb)
                prep_start(m + 3, nb)
            return _
        lax.fori_loop(0, 5, pipe_body, None)

        for m in range(20, n_chunks):
            process(m % 4)
            if m + 3 < n_chunks:
                wait_scatter((m + 3) % 4)
                prep_start(m + 3, (m + 3) % 4)

    for b in (1, 2, 3, 0):
        wait_scatter(b)
    plsc.subcore_barrier()

    def wout(i, _):
        k = sid + i * 16

        @pl.when(k < N // SUB)
        def _():
            pltpu.sync_copy(acc_sh.at[pl.ds(k * SUB, SUB)], rows_z)

            @pl.when(c == 0)
            def _():
                pltpu.sync_copy(rows_z, out0_hbm.at[pl.ds(k * SUB, SUB)])

            @pl.when(c == 1)
            def _():
                pltpu.sync_copy(rows_z, out1_hbm.at[pl.ds(k * SUB, SUB)])
        return _
    lax.fori_loop(0, 8, wout, None)


_BLK = 1000
_GRID = N // _BLK


def _tc_front_body(des_ref, tweet_ref, c0_ref, c1_ref, wd_ref, bd_ref, wt_ref,
                   bt_ref, wi_ref, bi_ref, wrel_ref, wroot_ref, brg_ref,
                   z_ref, root_ref, inv_ref):
    d = _leaky(jnp.dot(des_ref[...], wd_ref[...]) + bd_ref[...])
    t = _leaky(jnp.dot(tweet_ref[...], wt_ref[...]) + bt_ref[...])
    x0 = jnp.concatenate([d, t], axis=1)
    x = _leaky(jnp.dot(x0, wi_ref[...]) + bi_ref[...])
    root_ref[...] = jnp.dot(x, wroot_ref[...]) + brg_ref[...]
    for r in range(NUM_REL):
        z_ref[r] = jnp.dot(x, wrel_ref[r])
    cnt = c0_ref[...] + c1_ref[...]
    inv_ref[...] = 1.0 / jnp.maximum(cnt, 1.0)


def _tc_front(des, tweet, cnt0, cnt1, W_des, b_des, W_tweet, b_tweet, W_in,
              b_in, W_rel, W_root, b_rgcn):
    return pl.pallas_call(
        _tc_front_body,
        grid=(_GRID,),
        in_specs=[
            pl.BlockSpec((_BLK, 768), lambda i: (i, 0)),
            pl.BlockSpec((_BLK, 768), lambda i: (i, 0)),
            pl.BlockSpec((400, 128), lambda i: (0, 0)),
            pl.BlockSpec((400, 128), lambda i: (0, 0)),
            pl.BlockSpec((768, HALF), lambda i: (0, 0)),
            pl.BlockSpec((1, HALF), lambda i: (0, 0)),
            pl.BlockSpec((768, HALF), lambda i: (0, 0)),
            pl.BlockSpec((1, HALF), lambda i: (0, 0)),
            pl.BlockSpec((DIM, DIM), lambda i: (0, 0)),
            pl.BlockSpec((1, DIM), lambda i: (0, 0)),
            pl.BlockSpec((NUM_REL, DIM, DIM), lambda i: (0, 0, 0)),
            pl.BlockSpec((DIM, DIM), lambda i: (0, 0)),
            pl.BlockSpec((1, DIM), lambda i: (0, 0)),
        ],
        out_specs=[
            pl.BlockSpec((NUM_REL, _BLK, DIM), lambda i: (0, i, 0)),
            pl.BlockSpec((_BLK, DIM), lambda i: (i, 0)),
            pl.BlockSpec((400, 128), lambda i: (0, 0)),
        ],
        out_shape=[
            jax.ShapeDtypeStruct((NUM_REL, N, DIM), jnp.float32),
            jax.ShapeDtypeStruct((N, DIM), jnp.float32),
            jax.ShapeDtypeStruct((400, 128), jnp.float32),
        ],
    )(des, tweet, cnt0, cnt1, W_des, b_des, W_tweet, b_tweet, W_in, b_in,
      W_rel, W_root, b_rgcn)


def _tc_mid_body(root_ref, p0_ref, p1_ref, wrel_ref, wroot_ref, brg_ref,
                 z_ref, root2_ref):
    x = root_ref[...] + p0_ref[...] + p1_ref[...]
    root2_ref[...] = jnp.dot(x, wroot_ref[...]) + brg_ref[...]
    for r in range(NUM_REL):
        z_ref[r] = jnp.dot(x, wrel_ref[r])


def _tc_mid(root1, p0, p1, W_rel, W_root, b_rgcn):
    return pl.pallas_call(
        _tc_mid_body,
        grid=(_GRID,),
        in_specs=[
            pl.BlockSpec((_BLK, DIM), lambda i: (i, 0)),
            pl.BlockSpec((_BLK, DIM), lambda i: (i, 0)),
            pl.BlockSpec((_BLK, DIM), lambda i: (i, 0)),
            pl.BlockSpec((NUM_REL, DIM, DIM), lambda i: (0, 0, 0)),
            pl.BlockSpec((DIM, DIM), lambda i: (0, 0)),
            pl.BlockSpec((1, DIM), lambda i: (0, 0)),
        ],
        out_specs=[
            pl.BlockSpec((NUM_REL, _BLK, DIM), lambda i: (0, i, 0)),
            pl.BlockSpec((_BLK, DIM), lambda i: (i, 0)),
        ],
        out_shape=[
            jax.ShapeDtypeStruct((NUM_REL, N, DIM), jnp.float32),
            jax.ShapeDtypeStruct((N, DIM), jnp.float32),
        ],
    )(root1, p0, p1, W_rel, W_root, b_rgcn)


def _tc_final_body(root_ref, p0_ref, p1_ref, w1_ref, b1_ref, w2_ref, b2_ref,
                   y_ref):
    x = root_ref[...] + p0_ref[...] + p1_ref[...]
    h = _leaky(jnp.dot(x, w1_ref[...]) + b1_ref[...])
    y_ref[...] = jnp.dot(h, w2_ref[...]) + b2_ref[...]


def _tc_final(root2, p0, p1, W_out1, b_out1, W_out2p, b_out2p):
    return pl.pallas_call(
        _tc_final_body,
        grid=(_GRID,),
        in_specs=[
            pl.BlockSpec((_BLK, DIM), lambda i: (i, 0)),
            pl.BlockSpec((_BLK, DIM), lambda i: (i, 0)),
            pl.BlockSpec((_BLK, DIM), lambda i: (i, 0)),
            pl.BlockSpec((DIM, DIM), lambda i: (0, 0)),
            pl.BlockSpec((1, DIM), lambda i: (0, 0)),
            pl.BlockSpec((DIM, DIM), lambda i: (0, 0)),
            pl.BlockSpec((1, DIM), lambda i: (0, 0)),
        ],
        out_specs=pl.BlockSpec((_BLK, DIM), lambda i: (i, 0)),
        out_shape=jax.ShapeDtypeStruct((N, DIM), jnp.float32),
    )(root2, p0, p1, W_out1, b_out1, W_out2p, b_out2p)


def kernel(des, tweet, num_prop, cat_prop, edge_index, edge_type,
           W_des, b_des, W_tweet, b_tweet, W_in, b_in,
           W_rel, W_root, b_rgcn, W_out1, b_out1, W_out2, b_out2):
    src = edge_index[0]
    dst = edge_index[1]
    typ = edge_type

    cnt0, cnt1 = _sc_counts(dst, typ)

    z1, root1, invr = _tc_front(
        des, tweet, cnt0.reshape(400, 128), cnt1.reshape(400, 128), W_des,
        b_des.reshape(1, -1), W_tweet, b_tweet.reshape(1, -1), W_in,
        b_in.reshape(1, -1), W_rel, W_root, b_rgcn.reshape(1, -1))
    inv = invr.reshape(RN_PAD)

    p0, p1 = _sc_agg(z1.reshape(RN, DIM), inv, src, dst, typ)
    z2, root2 = _tc_mid(root1, p0, p1, W_rel, W_root, b_rgcn.reshape(1, -1))
    q0, q1 = _sc_agg(z2.reshape(RN, DIM), inv, src, dst, typ)

    W_out2p = jnp.zeros((DIM, DIM), jnp.float32).at[:, :2].set(W_out2)
    b_out2p = jnp.zeros((1, DIM), jnp.float32).at[0, :2].set(b_out2)
    y = _tc_final(root2, q0, q1, W_out1, b_out1.reshape(1, -1), W_out2p,
                  b_out2p)
    return y[:, :2]


# speedup vs baseline: 1.0279x; 1.0279x over previous
import functools

import jax
import jax.numpy as jnp
from jax import lax
from jax.experimental import pallas as pl
from jax.experimental.pallas import tpu as pltpu
from jax.experimental.pallas import tpu_sc as plsc

N = 10000
E = 320000
NUM_REL = 5
DIM = 128
HALF = DIM // 2
RN = NUM_REL * N
RN_PAD = 51200

NUM_TILES = 32
EPT = E // NUM_TILES
SUP = 2000
SUB = 80
N_PER_SUB = N // 16

_mesh = plsc.VectorSubcoreMesh(core_axis_name="c", subcore_axis_name="s")


def _leaky(x):
    return jnp.where(x > 0, x, 0.01 * x)


@functools.partial(
    pl.kernel,
    out_type=[jax.ShapeDtypeStruct((RN_PAD,), jnp.float32),
              jax.ShapeDtypeStruct((RN_PAD,), jnp.float32)],
    mesh=_mesh,
    compiler_params=pltpu.CompilerParams(needs_layout_passes=False),
    scratch_types=[
        pltpu.VMEM_SHARED((RN_PAD,), jnp.float32),
        pltpu.VMEM((SUP,), jnp.int32),
        pltpu.VMEM((SUP,), jnp.int32),
        pltpu.VMEM((SUB,), jnp.int32),
        pltpu.VMEM((SUB,), jnp.int32),
        pltpu.VMEM((SUB,), jnp.int32),
        pltpu.VMEM((SUB,), jnp.int32),
        pltpu.VMEM((SUB,), jnp.float32),
        pltpu.VMEM((3200,), jnp.float32),
        pltpu.SemaphoreType.DMA,
        pltpu.SemaphoreType.DMA,
        pltpu.SemaphoreType.DMA,
        pltpu.SemaphoreType.DMA,
    ],
)
def _sc_counts(dst_hbm, typ_hbm, out0_hbm, out1_hbm, acc_sh, dst_v, typ_v,
               key_v0, key_v1, key_v2, key_v3, one_v, buf_v,
               ssem0, ssem1, ssem2, ssem3):
    c = lax.axis_index("c")
    sid = lax.axis_index("s")
    wid = sid * 2 + c
    keys = (key_v0, key_v1, key_v2, key_v3)
    ssems = (ssem0, ssem1, ssem2, ssem3)

    def zbuf(i, _):
        buf_v[pl.ds(i * 16, 16)] = jnp.zeros((16,), jnp.float32)
        return _
    lax.fori_loop(0, 200, zbuf, None)
    pltpu.sync_copy(buf_v, acc_sh.at[pl.ds(sid * 3200, 3200)])

    def ones(i, _):
        one_v[pl.ds(i * 16, 16)] = jnp.ones((16,), jnp.float32)
        return _
    lax.fori_loop(0, SUB // 16, ones, None)
    plsc.subcore_barrier()

    ebase = wid * EPT
    n_chunks = SUP // SUB

    def wait_scatter(b):
        pltpu.make_async_copy(one_v, acc_sh.at[keys[b]], ssems[b]).wait()

    def launch(m, b):
        for j in range(SUB // 16):
            off = m * SUB + j * 16
            d16 = dst_v[pl.ds(off, 16)]
            t16 = typ_v[pl.ds(off, 16)]
            keys[b][pl.ds(j * 16, 16)] = t16 * N + d16
        pltpu.async_copy(one_v, acc_sh.at[keys[b]], ssems[b], add=True)

    for sup in range(EPT // SUP):
        base = ebase + sup * SUP
        pltpu.sync_copy(dst_hbm.at[pl.ds(base, SUP)], dst_v)
        pltpu.sync_copy(typ_hbm.at[pl.ds(base, SUP)], typ_v)

        def pipe_body(j, _, _first_sup=(sup == 0)):
            for rr in range(4):
                m = 4 * j + rr
                if _first_sup:
                    @pl.when(j > 0)
                    def _():
                        wait_scatter(rr)
                else:
                    wait_scatter(rr)
                launch(m, rr)
            return _
        lax.fori_loop(0, 6, pipe_body, None)
        wait_scatter(0)
        launch(n_chunks - 1, 0)

    for b in (1, 2, 3, 0):
        wait_scatter(b)
    plsc.subcore_barrier()

    pltpu.sync_copy(acc_sh.at[pl.ds(sid * 3200, 3200)], buf_v)

    @pl.when(c == 0)
    def _():
        pltpu.sync_copy(buf_v, out0_hbm.at[pl.ds(sid * 3200, 3200)])

    @pl.when(c == 1)
    def _():
        pltpu.sync_copy(buf_v, out1_hbm.at[pl.ds(sid * 3200, 3200)])


NBUF = 4


def _agg_scratch():
    per_buf = []
    for _ in range(NBUF):
        per_buf += [
            pltpu.VMEM((SUB,), jnp.int32),
            pltpu.VMEM((SUB,), jnp.int32),
            pltpu.VMEM((SUB,), jnp.int32),
            pltpu.VMEM((SUB,), jnp.float32),
            pltpu.VMEM((SUB, DIM), jnp.float32),
            pltpu.SemaphoreType.DMA,
            pltpu.SemaphoreType.DMA,
        ]
    return [
        pltpu.VMEM_SHARED((N, DIM), jnp.float32),
        pltpu.VMEM((SUP,), jnp.int32),
        pltpu.VMEM((SUP,), jnp.int32),
        pltpu.VMEM((SUP,), jnp.int32),
    ] + per_buf


@functools.partial(
    pl.kernel,
    out_type=[jax.ShapeDtypeStruct((N, DIM), jnp.float32),
              jax.ShapeDtypeStruct((N, DIM), jnp.float32)],
    mesh=_mesh,
    compiler_params=pltpu.CompilerParams(needs_layout_passes=False),
    scratch_types=_agg_scratch(),
)
def _sc_agg(zt_hbm, inv_hbm, src_hbm, dst_hbm, typ_hbm, out0_hbm, out1_hbm,
            acc_sh, src_v, dst_v, typ_v, *bufflat):
    c = lax.axis_index("c")
    sid = lax.axis_index("s")
    wid = sid * 2 + c
    bufs = tuple(bufflat[i * 7:(i + 1) * 7] for i in range(NBUF))
    rows_z = bufs[0][4]

    def zrow16(i, _):
        rows_z[i // 8, pl.ds((i % 8) * 16, 16)] = jnp.zeros((16,), jnp.float32)
        return _
    lax.fori_loop(0, SUB * 8, zrow16, None)

    def zacc(i, _):
        k = sid + i * 16

        @pl.when(k < N // SUB)
        def _():
            pltpu.sync_copy(rows_z, acc_sh.at[pl.ds(k * SUB, SUB)])
        return _
    lax.fori_loop(0, 8, zacc, None)
    plsc.subcore_barrier()

    ebase = wid * EPT
    n_chunks = SUP // SUB

    def wait_scatter(b):
        _, _, d_v, _, rows_v, _, ssem = bufs[b]
        pltpu.make_async_copy(rows_v, acc_sh.at[d_v], ssem).wait()

    def prep_start(m, b):
        g_v, k_v, d_v, w_v, rows_v, gsem, _ = bufs[b]

        for j in range(SUB // 16):
            off = m * SUB + j * 16
            s16 = src_v[pl.ds(off, 16)]
            d16 = dst_v[pl.ds(off, 16)]
            t16 = typ_v[pl.ds(off, 16)]
            g_v[pl.ds(j * 16, 16)] = t16 * N + s16
            d_v[pl.ds(j * 16, 16)] = d16
            k_v[pl.ds(j * 16, 16)] = t16 * N + d16
        pltpu.async_copy(inv_hbm.at[k_v], w_v, gsem)
        pltpu.async_copy(zt_hbm.at[g_v], rows_v, gsem)

    def process(b):
        g_v, k_v, d_v, w_v, rows_v, gsem, ssem = bufs[b]
        pltpu.make_async_copy(inv_hbm.at[k_v], w_v, gsem).wait()
        pltpu.make_async_copy(zt_hbm.at[g_v], rows_v, gsem).wait()

        def scale_body(ii, _):
            for rr in range(4):
                i = ii * 4 + rr
                wb = plsc.load_gather(w_v, [jnp.broadcast_to(i, (16,))])
                for jj in range(DIM // 16):
                    sl = pl.ds(jj * 16, 16)
                    rows_v[i, sl] = rows_v[i, sl] * wb
            return _
        lax.fori_loop(0, SUB // 4, scale_body, None)
        pltpu.async_copy(rows_v, acc_sh.at[d_v], ssem, add=True)

    for sup in range(EPT // SUP):
        base = ebase + sup * SUP
        pltpu.sync_copy(src_hbm.at[pl.ds(base, SUP)], src_v)
        pltpu.sync_copy(dst_hbm.at[pl.ds(base, SUP)], dst_v)
        pltpu.sync_copy(typ_hbm.at[pl.ds(base, SUP)], typ_v)

        for b in range(NBUF - 1):
            if sup > 0:
                wait_scatter(b)
            prep_start(b, b)

        def pipe_body(j, _, _first_sup=(sup == 0)):
            for rr in range(4):
                m = 4 * j + rr
                process(rr)
                nb = (rr + 3) % 4
                if _first_sup and rr == 0:
                    @pl.when(j > 0)
                    def _():
                        wait_scatter(nb)
                else:
                    wait_scatter(nb)
                prep_start(m + 3, nb)
            return _
        lax.fori_loop(0, 5, pipe_body, None)

        for m in range(20, n_chunks):
            process(m % 4)
            if m + 3 < n_chunks:
                wait_scatter((m + 3) % 4)
                prep_start(m + 3, (m + 3) % 4)

    for b in (1, 2, 3, 0):
        wait_scatter(b)
    plsc.subcore_barrier()

    def wout(i, _):
        k = sid + i * 16

        @pl.when(k < N // SUB)
        def _():
            pltpu.sync_copy(acc_sh.at[pl.ds(k * SUB, SUB)], rows_z)

            @pl.when(c == 0)
            def _():
                pltpu.sync_copy(rows_z, out0_hbm.at[pl.ds(k * SUB, SUB)])

            @pl.when(c == 1)
            def _():
                pltpu.sync_copy(rows_z, out1_hbm.at[pl.ds(k * SUB, SUB)])
        return _
    lax.fori_loop(0, 8, wout, None)


_BLK = 1000
_GRID = N // _BLK


def _tc_front_body(des_ref, tweet_ref, c0_ref, c1_ref, wd_ref, bd_ref, wt_ref,
                   bt_ref, wi_ref, bi_ref, wrel_ref, wroot_ref, brg_ref,
                   z_ref, root_ref, inv_ref):
    d = _leaky(jnp.dot(des_ref[...], wd_ref[...]) + bd_ref[...])
    t = _leaky(jnp.dot(tweet_ref[...], wt_ref[...]) + bt_ref[...])
    x0 = jnp.concatenate([d, t], axis=1)
    x = _leaky(jnp.dot(x0, wi_ref[...]) + bi_ref[...])
    root_ref[...] = jnp.dot(x, wroot_ref[...]) + brg_ref[...]
    for r in range(NUM_REL):
        z_ref[r] = jnp.dot(x, wrel_ref[r])
    cnt = c0_ref[...] + c1_ref[...]
    inv_ref[...] = 1.0 / jnp.maximum(cnt, 1.0)


def _tc_front(des, tweet, cnt0, cnt1, W_des, b_des, W_tweet, b_tweet, W_in,
              b_in, W_rel, W_root, b_rgcn):
    return pl.pallas_call(
        _tc_front_body,
        grid=(_GRID,),
        in_specs=[
            pl.BlockSpec((_BLK, 768), lambda i: (i, 0)),
            pl.BlockSpec((_BLK, 768), lambda i: (i, 0)),
            pl.BlockSpec((400, 128), lambda i: (0, 0)),
            pl.BlockSpec((400, 128), lambda i: (0, 0)),
            pl.BlockSpec((768, HALF), lambda i: (0, 0)),
            pl.BlockSpec((1, HALF), lambda i: (0, 0)),
            pl.BlockSpec((768, HALF), lambda i: (0, 0)),
            pl.BlockSpec((1, HALF), lambda i: (0, 0)),
            pl.BlockSpec((DIM, DIM), lambda i: (0, 0)),
            pl.BlockSpec((1, DIM), lambda i: (0, 0)),
            pl.BlockSpec((NUM_REL, DIM, DIM), lambda i: (0, 0, 0)),
            pl.BlockSpec((DIM, DIM), lambda i: (0, 0)),
            pl.BlockSpec((1, DIM), lambda i: (0, 0)),
        ],
        out_specs=[
            pl.BlockSpec((NUM_REL, _BLK, DIM), lambda i: (0, i, 0)),
            pl.BlockSpec((_BLK, DIM), lambda i: (i, 0)),
            pl.BlockSpec((400, 128), lambda i: (0, 0)),
        ],
        out_shape=[
            jax.ShapeDtypeStruct((NUM_REL, N, DIM), jnp.float32),
            jax.ShapeDtypeStruct((N, DIM), jnp.float32),
            jax.ShapeDtypeStruct((400, 128), jnp.float32),
        ],
    )(des, tweet, cnt0, cnt1, W_des, b_des, W_tweet, b_tweet, W_in, b_in,
      W_rel, W_root, b_rgcn)


def _tc_mid_body(root_ref, p0_ref, p1_ref, wrel_ref, wroot_ref, brg_ref,
                 z_ref, root2_ref):
    x = root_ref[...] + p0_ref[...] + p1_ref[...]
    root2_ref[...] = jnp.dot(x, wroot_ref[...]) + brg_ref[...]
    for r in range(NUM_REL):
        z_ref[r] = jnp.dot(x, wrel_ref[r])


def _tc_mid(root1, p0, p1, W_rel, W_root, b_rgcn):
    return pl.pallas_call(
        _tc_mid_body,
        grid=(_GRID,),
        in_specs=[
            pl.BlockSpec((_BLK, DIM), lambda i: (i, 0)),
            pl.BlockSpec((_BLK, DIM), lambda i: (i, 0)),
            pl.BlockSpec((_BLK, DIM), lambda i: (i, 0)),
            pl.BlockSpec((NUM_REL, DIM, DIM), lambda i: (0, 0, 0)),
            pl.BlockSpec((DIM, DIM), lambda i: (0, 0)),
            pl.BlockSpec((1, DIM), lambda i: (0, 0)),
        ],
        out_specs=[
            pl.BlockSpec((NUM_REL, _BLK, DIM), lambda i: (0, i, 0)),
            pl.BlockSpec((_BLK, DIM), lambda i: (i, 0)),
        ],
        out_shape=[
            jax.ShapeDtypeStruct((NUM_REL, N, DIM), jnp.float32),
            jax.ShapeDtypeStruct((N, DIM), jnp.float32),
        ],
    )(root1, p0, p1, W_rel, W_root, b_rgcn)


def _tc_final_body(root_ref, p0_ref, p1_ref, w1_ref, b1_ref, w2_ref, b2_ref,
                   y_ref):
    x = root_ref[...] + p0_ref[...] + p1_ref[...]
    h = _leaky(jnp.dot(x, w1_ref[...]) + b1_ref[...])
    y_ref[...] = jnp.dot(h, w2_ref[...]) + b2_ref[...]


def _tc_final(root2, p0, p1, W_out1, b_out1, W_out2p, b_out2p):
    return pl.pallas_call(
        _tc_final_body,
        grid=(_GRID,),
        in_specs=[
            pl.BlockSpec((_BLK, DIM), lambda i: (i, 0)),
            pl.BlockSpec((_BLK, DIM), lambda i: (i, 0)),
            pl.BlockSpec((_BLK, DIM), lambda i: (i, 0)),
            pl.BlockSpec((DIM, DIM), lambda i: (0, 0)),
            pl.BlockSpec((1, DIM), lambda i: (0, 0)),
            pl.BlockSpec((DIM, DIM), lambda i: (0, 0)),
            pl.BlockSpec((1, DIM), lambda i: (0, 0)),
        ],
        out_specs=pl.BlockSpec((_BLK, DIM), lambda i: (i, 0)),
        out_shape=jax.ShapeDtypeStruct((N, DIM), jnp.float32),
    )(root2, p0, p1, W_out1, b_out1, W_out2p, b_out2p)


def kernel(des, tweet, num_prop, cat_prop, edge_index, edge_type,
           W_des, b_des, W_tweet, b_tweet, W_in, b_in,
           W_rel, W_root, b_rgcn, W_out1, b_out1, W_out2, b_out2):
    src = edge_index[0]
    dst = edge_index[1]
    typ = edge_type

    cnt0, cnt1 = _sc_counts(dst, typ)

    z1, root1, invr = _tc_front(
        des, tweet, cnt0.reshape(400, 128), cnt1.reshape(400, 128), W_des,
        b_des.reshape(1, -1), W_tweet, b_tweet.reshape(1, -1), W_in,
        b_in.reshape(1, -1), W_rel, W_root, b_rgcn.reshape(1, -1))
    inv = invr.reshape(RN_PAD)

    p0, p1 = _sc_agg(z1.reshape(RN, DIM), inv, src, dst, typ)
    z2, root2 = _tc_mid(root1, p0, p1, W_rel, W_root, b_rgcn.reshape(1, -1))
    q0, q1 = _sc_agg(z2.reshape(RN, DIM), inv, src, dst, typ)

    W_out2p = jnp.zeros((DIM, DIM), jnp.float32).at[:, :2].set(W_out2)
    b_out2p = jnp.zeros((1, DIM), jnp.float32).at[0, :2].set(b_out2)
    y = _tc_final(root2, q0, q1, W_out1, b_out1.reshape(1, -1), W_out2p,
                  b_out2p)
    return y[:, :2]
